# Initial kernel scaffold; baseline (speedup 1.0000x reference)
import jax
import jax.numpy as jnp
from jax.experimental import pallas as pl


def kernel(x, edge_index, batch, W1, b1, W2, b2, W3, b3):
    raise NotImplementedError("write your pallas kernel here")


# speedup vs baseline: 24.0093x; 24.0093x over previous
import functools

import jax
import jax.numpy as jnp
from jax import lax
from jax.experimental import pallas as pl
from jax.experimental.pallas import tpu as pltpu
from jax.experimental.pallas import tpu_sc as plsc

N = 10000
E = 640000
G = 64
D_IN = 121
D_H1 = 16
D_H2 = 8

NC = 2
NS = 16
NW = NC * NS
CHUNK = 128
NCHUNKS = E // CHUNK
CHUNKS_PER_W = -(-NCHUNKS // NW)

_sc_mesh = plsc.VectorSubcoreMesh(core_axis_name="c", subcore_axis_name="s")


@functools.partial(
    pl.kernel,
    out_type=jax.ShapeDtypeStruct((NC, N, D_H1), jnp.float32),
    mesh=_sc_mesh,
    scratch_types=[
        pltpu.VMEM((CHUNK,), jnp.int32),
        pltpu.VMEM((CHUNK,), jnp.int32),
        pltpu.VMEM((CHUNK, D_H1), jnp.float32),
        pltpu.VMEM_SHARED((N, D_H1), jnp.float32),
        pltpu.SemaphoreType.DMA,
    ],
    compiler_params=pltpu.CompilerParams(use_tc_tiling_on_sc=False),
)
def _sc_edge_agg(g_hbm, src_hbm, dst_hbm, zero_hbm, out_hbm,
                 sidx, didx, rows, acc, sem):
    cid = lax.axis_index("c")
    sid = lax.axis_index("s")
    wid = sid * NC + cid

    @pl.when(sid == 0)
    def _():
        pltpu.sync_copy(zero_hbm, acc)

    plsc.subcore_barrier()

    def body(k, carry):
        chunk = k * NW + wid

        @pl.when(chunk < NCHUNKS)
        def _():
            base = chunk * CHUNK
            pltpu.sync_copy(src_hbm.at[pl.ds(base, CHUNK)], sidx)
            pltpu.sync_copy(dst_hbm.at[pl.ds(base, CHUNK)], didx)
            pltpu.async_copy(g_hbm.at[sidx], rows, sem).wait()
            pltpu.sync_copy(rows, acc.at[didx], add=True)

        return carry

    lax.fori_loop(0, CHUNKS_PER_W, body, 0)
    plsc.subcore_barrier()

    @pl.when(sid == 0)
    def _():
        pltpu.sync_copy(acc, out_hbm.at[cid])


@functools.partial(
    pl.kernel,
    out_type=jax.ShapeDtypeStruct((NC, N, D_H1), jnp.float32),
    mesh=_sc_mesh,
    scratch_types=[
        pltpu.VMEM((CHUNK,), jnp.int32),
        pltpu.VMEM((CHUNK, D_H1), jnp.float32),
        pltpu.VMEM_SHARED((N, D_H1), jnp.float32),
    ],
    compiler_params=pltpu.CompilerParams(use_tc_tiling_on_sc=False),
)
def _sc_degree(dst_hbm, ones_hbm, zero_hbm, out_hbm, didx, ones_v, acc):
    cid = lax.axis_index("c")
    sid = lax.axis_index("s")
    wid = sid * NC + cid

    @pl.when(sid == 0)
    def _():
        pltpu.sync_copy(zero_hbm, acc)

    pltpu.sync_copy(ones_hbm, ones_v)
    plsc.subcore_barrier()

    def body(k, carry):
        chunk = k * NW + wid

        @pl.when(chunk < NCHUNKS)
        def _():
            base = chunk * CHUNK
            pltpu.sync_copy(dst_hbm.at[pl.ds(base, CHUNK)], didx)
            pltpu.sync_copy(ones_v, acc.at[didx], add=True)

        return carry

    lax.fori_loop(0, CHUNKS_PER_W, body, 0)
    plsc.subcore_barrier()

    @pl.when(sid == 0)
    def _():
        pltpu.sync_copy(acc, out_hbm.at[cid])


def _tc_project1(x_ref, w1_ref, p1_ref):
    p1_ref[...] = jnp.dot(x_ref[...], w1_ref[...],
                          preferred_element_type=jnp.float32)


def _tc_scale1(degp_ref, p1_ref, dinv_ref, g1_ref):
    deg = degp_ref[0, :, :1] + degp_ref[1, :, :1] + 1.0
    dinv = lax.rsqrt(deg)
    dinv_ref[...] = dinv
    g1_ref[...] = dinv * p1_ref[...]


def _tc_mid(aggp_ref, p1_ref, dinv_ref, b1_ref, w2_ref, g2_ref, p2_ref):
    dinv = dinv_ref[...]
    agg = aggp_ref[0] + aggp_ref[1]
    h1 = jnp.maximum(dinv * agg + (dinv * dinv) * p1_ref[...] + b1_ref[...],
                     0.0)
    p2 = jnp.dot(h1, w2_ref[...], preferred_element_type=jnp.float32)
    p2_ref[...] = p2
    g2_ref[...] = jnp.concatenate(
        [dinv * p2, jnp.zeros((N, D_H1 - D_H2), jnp.float32)], axis=1)


def _tc_tail(aggp_ref, p2_ref, dinv_ref, b2_ref, batch_ref, ones_ref,
             w3_ref, b3_ref, out_ref):
    dinv = dinv_ref[...]
    agg = (aggp_ref[0] + aggp_ref[1])[:, :D_H2]
    h2 = jnp.maximum(dinv * agg + (dinv * dinv) * p2_ref[...] + b2_ref[...],
                     0.0)
    onehot_t = (lax.broadcasted_iota(jnp.int32, (G, N), 0)
                == batch_ref[...]).astype(jnp.float32)
    sums = jnp.dot(onehot_t, h2, preferred_element_type=jnp.float32)
    counts = jnp.dot(onehot_t, ones_ref[...],
                     preferred_element_type=jnp.float32)
    pooled = sums / jnp.maximum(counts, 1.0)
    out_ref[...] = jnp.dot(pooled, w3_ref[...],
                           preferred_element_type=jnp.float32) + b3_ref[...]


def kernel(x, edge_index, batch, W1, b1, W2, b2, W3, b3):
    src = edge_index[0]
    dst = edge_index[1]
    zero16 = jnp.zeros((N, D_H1), jnp.float32)
    ones_chunk = jnp.ones((CHUNK, D_H1), jnp.float32)
    ones_col = jnp.ones((N, 1), jnp.float32)
    batch_row = batch.reshape(1, N)
    b1_2d = b1.reshape(1, D_H1)
    b2_2d = b2.reshape(1, D_H2)
    b3_2d = b3.reshape(1, 1)

    degp = _sc_degree(dst, ones_chunk, zero16)

    p1 = pl.pallas_call(
        _tc_project1,
        out_shape=jax.ShapeDtypeStruct((N, D_H1), jnp.float32),
    )(x, W1)

    dinv, g1 = pl.pallas_call(
        _tc_scale1,
        out_shape=(jax.ShapeDtypeStruct((N, 1), jnp.float32),
                   jax.ShapeDtypeStruct((N, D_H1), jnp.float32)),
    )(degp, p1)

    agg1p = _sc_edge_agg(g1, src, dst, zero16)

    g2, p2 = pl.pallas_call(
        _tc_mid,
        out_shape=(jax.ShapeDtypeStruct((N, D_H1), jnp.float32),
                   jax.ShapeDtypeStruct((N, D_H2), jnp.float32)),
    )(agg1p, p1, dinv, b1_2d, W2)

    agg2p = _sc_edge_agg(g2, src, dst, zero16)

    out = pl.pallas_call(
        _tc_tail,
        out_shape=jax.ShapeDtypeStruct((G, 1), jnp.float32),
    )(agg2p, p2, dinv, b2_2d, batch_row, ones_col, W3, b3_2d)

    return out


# speedup vs baseline: 46.2147x; 1.9249x over previous
import functools

import jax
import jax.numpy as jnp
from jax import lax
from jax.experimental import pallas as pl
from jax.experimental.pallas import tpu as pltpu
from jax.experimental.pallas import tpu_sc as plsc

N = 10000
E = 640000
G = 64
D_IN = 121
D_H1 = 16
D_H2 = 8

NC = 2
NS = 16
NW = NC * NS
CHUNK = 128
NKC = 160
E_PAD = NW * NKC * CHUNK
NPAD = N + 8
NBUF = 4
AHEAD = 2

_sc_mesh = plsc.VectorSubcoreMesh(core_axis_name="c", subcore_axis_name="s")


@functools.partial(
    pl.kernel,
    out_type=jax.ShapeDtypeStruct((NC, N, D_H1), jnp.float32),
    mesh=_sc_mesh,
    scratch_types=[
        pltpu.VMEM((NKC, CHUNK), jnp.int32),
        pltpu.VMEM((NKC, CHUNK), jnp.int32),
        pltpu.VMEM((CHUNK, D_H1), jnp.float32),
        pltpu.VMEM((CHUNK, D_H1), jnp.float32),
        pltpu.VMEM((CHUNK, D_H1), jnp.float32),
        pltpu.VMEM((CHUNK, D_H1), jnp.float32),
        pltpu.VMEM_SHARED((NPAD, D_H1), jnp.float32),
        pltpu.SemaphoreType.DMA,
        pltpu.SemaphoreType.DMA,
        pltpu.SemaphoreType.DMA,
        pltpu.SemaphoreType.DMA,
        pltpu.SemaphoreType.DMA,
        pltpu.SemaphoreType.DMA,
        pltpu.SemaphoreType.DMA,
        pltpu.SemaphoreType.DMA,
    ],
    compiler_params=pltpu.CompilerParams(use_tc_tiling_on_sc=False),
)
def _sc_edge_agg(g_hbm, src_hbm, dst_hbm, zero_hbm, out_hbm,
                 sidx, didx, r0, r1, r2, r3, acc,
                 gs0, gs1, gs2, gs3, ss0, ss1, ss2, ss3):
    cid = lax.axis_index("c")
    sid = lax.axis_index("s")
    wid = sid * NC + cid
    rows = (r0, r1, r2, r3)
    gsem = (gs0, gs1, gs2, gs3)
    ssem = (ss0, ss1, ss2, ss3)

    @pl.when(sid == 0)
    def _():
        pltpu.sync_copy(zero_hbm, acc)

    base = wid * NKC
    pltpu.sync_copy(src_hbm.at[pl.ds(base, NKC)], sidx)
    pltpu.sync_copy(dst_hbm.at[pl.ds(base, NKC)], didx)
    plsc.subcore_barrier()

    for k in range(AHEAD):
        pltpu.async_copy(g_hbm.at[sidx.at[k]], rows[k % NBUF], gsem[k % NBUF])

    def body(i, carry):
        for j in range(NBUF):
            k = i * NBUF + j
            pltpu.make_async_copy(g_hbm.at[sidx.at[k]], rows[j],
                                  gsem[j]).wait()
            pltpu.async_copy(rows[j], acc.at[didx.at[k]], ssem[j], add=True)
            m = k + AHEAD
            jj = (j + AHEAD) % NBUF

            @pl.when((m >= NBUF) & (m < NKC))
            def _():
                pltpu.make_async_copy(rows[jj], acc.at[didx.at[0]],
                                      ssem[jj]).wait()

            @pl.when(m < NKC)
            def _():
                pltpu.async_copy(g_hbm.at[sidx.at[m]], rows[jj], gsem[jj])

        return carry

    lax.fori_loop(0, NKC // NBUF, body, 0)

    for j in range(NBUF):
        pltpu.make_async_copy(rows[j], acc.at[didx.at[0]], ssem[j]).wait()

    plsc.subcore_barrier()

    @pl.when(sid == 0)
    def _():
        pltpu.sync_copy(acc.at[pl.ds(0, N)], out_hbm.at[cid])


@functools.partial(
    pl.kernel,
    out_type=jax.ShapeDtypeStruct((NC, N, D_H1), jnp.float32),
    mesh=_sc_mesh,
    scratch_types=[
        pltpu.VMEM((NKC, CHUNK), jnp.int32),
        pltpu.VMEM((CHUNK, D_H1), jnp.float32),
        pltpu.VMEM_SHARED((NPAD, D_H1), jnp.float32),
        pltpu.SemaphoreType.DMA,
        pltpu.SemaphoreType.DMA,
        pltpu.SemaphoreType.DMA,
        pltpu.SemaphoreType.DMA,
    ],
    compiler_params=pltpu.CompilerParams(use_tc_tiling_on_sc=False),
)
def _sc_degree(dst_hbm, ones_hbm, zero_hbm, out_hbm, didx, ones_v, acc,
               ss0, ss1, ss2, ss3):
    cid = lax.axis_index("c")
    sid = lax.axis_index("s")
    wid = sid * NC + cid
    ssem = (ss0, ss1, ss2, ss3)

    @pl.when(sid == 0)
    def _():
        pltpu.sync_copy(zero_hbm, acc)

    pltpu.sync_copy(ones_hbm, ones_v)
    pltpu.sync_copy(dst_hbm.at[pl.ds(wid * NKC, NKC)], didx)
    plsc.subcore_barrier()

    def body(i, carry):
        for j in range(NBUF):
            k = i * NBUF + j

            @pl.when(k >= NBUF)
            def _():
                pltpu.make_async_copy(ones_v, acc.at[didx.at[0]],
                                      ssem[j]).wait()

            pltpu.async_copy(ones_v, acc.at[didx.at[k]], ssem[j], add=True)
        return carry

    lax.fori_loop(0, NKC // NBUF, body, 0)

    for j in range(NBUF):
        pltpu.make_async_copy(ones_v, acc.at[didx.at[0]], ssem[j]).wait()

    plsc.subcore_barrier()

    @pl.when(sid == 0)
    def _():
        pltpu.sync_copy(acc.at[pl.ds(0, N)], out_hbm.at[cid])


def _tc_project1(x_ref, w1_ref, p1_ref):
    p1_ref[...] = jnp.dot(x_ref[...], w1_ref[...],
                          preferred_element_type=jnp.float32)


def _tc_scale1(degp_ref, p1_ref, dinv_ref, g1_ref):
    deg = degp_ref[0, :, :1] + degp_ref[1, :, :1] + 1.0
    dinv = lax.rsqrt(deg)
    dinv_ref[...] = dinv
    g1_ref[...] = dinv * p1_ref[...]


def _tc_mid(aggp_ref, p1_ref, dinv_ref, b1_ref, w2_ref, g2_ref, p2_ref):
    dinv = dinv_ref[...]
    agg = aggp_ref[0] + aggp_ref[1]
    h1 = jnp.maximum(dinv * agg + (dinv * dinv) * p1_ref[...] + b1_ref[...],
                     0.0)
    p2 = jnp.dot(h1, w2_ref[...], preferred_element_type=jnp.float32)
    p2_ref[...] = p2
    g2_ref[...] = jnp.concatenate(
        [dinv * p2, jnp.zeros((N, D_H1 - D_H2), jnp.float32)], axis=1)


def _tc_tail(aggp_ref, p2_ref, dinv_ref, b2_ref, batch_ref, ones_ref,
             w3_ref, b3_ref, out_ref):
    dinv = dinv_ref[...]
    agg = (aggp_ref[0] + aggp_ref[1])[:, :D_H2]
    h2 = jnp.maximum(dinv * agg + (dinv * dinv) * p2_ref[...] + b2_ref[...],
                     0.0)
    onehot_t = (lax.broadcasted_iota(jnp.int32, (G, N), 0)
                == batch_ref[...]).astype(jnp.float32)
    sums = jnp.dot(onehot_t, h2, preferred_element_type=jnp.float32)
    counts = jnp.dot(onehot_t, ones_ref[...],
                     preferred_element_type=jnp.float32)
    pooled = sums / jnp.maximum(counts, 1.0)
    out_ref[...] = jnp.dot(pooled, w3_ref[...],
                           preferred_element_type=jnp.float32) + b3_ref[...]


def kernel(x, edge_index, batch, W1, b1, W2, b2, W3, b3):
    npad_e = E_PAD - E
    src2 = jnp.concatenate(
        [edge_index[0], jnp.zeros((npad_e,), jnp.int32)]).reshape(-1, CHUNK)
    dst2 = jnp.concatenate(
        [edge_index[1], jnp.full((npad_e,), N, jnp.int32)]).reshape(-1, CHUNK)
    zero16 = jnp.zeros((NPAD, D_H1), jnp.float32)
    ones_chunk = jnp.ones((CHUNK, D_H1), jnp.float32)
    ones_col = jnp.ones((N, 1), jnp.float32)
    batch_row = batch.reshape(1, N)
    b1_2d = b1.reshape(1, D_H1)
    b2_2d = b2.reshape(1, D_H2)
    b3_2d = b3.reshape(1, 1)

    degp = _sc_degree(dst2, ones_chunk, zero16)

    p1 = pl.pallas_call(
        _tc_project1,
        out_shape=jax.ShapeDtypeStruct((N, D_H1), jnp.float32),
    )(x, W1)

    dinv, g1 = pl.pallas_call(
        _tc_scale1,
        out_shape=(jax.ShapeDtypeStruct((N, 1), jnp.float32),
                   jax.ShapeDtypeStruct((N, D_H1), jnp.float32)),
    )(degp, p1)

    agg1p = _sc_edge_agg(g1, src2, dst2, zero16)

    g2, p2 = pl.pallas_call(
        _tc_mid,
        out_shape=(jax.ShapeDtypeStruct((N, D_H1), jnp.float32),
                   jax.ShapeDtypeStruct((N, D_H2), jnp.float32)),
    )(agg1p, p1, dinv, b1_2d, W2)

    agg2p = _sc_edge_agg(g2, src2, dst2, zero16)

    out = pl.pallas_call(
        _tc_tail,
        out_shape=jax.ShapeDtypeStruct((G, 1), jnp.float32),
    )(agg2p, p2, dinv, b2_2d, batch_row, ones_col, W3, b3_2d)

    return out


# speedup vs baseline: 46.4459x; 1.0050x over previous
import functools

import jax
import jax.numpy as jnp
from jax import lax
from jax.experimental import pallas as pl
from jax.experimental.pallas import tpu as pltpu
from jax.experimental.pallas import tpu_sc as plsc

N = 10000
E = 640000
G = 64
D_IN = 121
D_H1 = 16
D_H2 = 8

NC = 2
NS = 16
NW = NC * NS
CHUNK = 128
NKC = 160
E_PAD = NW * NKC * CHUNK
NPAD = N + 8
NBUF = 8
AHEAD = 6

_sc_mesh = plsc.VectorSubcoreMesh(core_axis_name="c", subcore_axis_name="s")


@functools.partial(
    pl.kernel,
    out_type=jax.ShapeDtypeStruct((NC, N, D_H1), jnp.float32),
    mesh=_sc_mesh,
    scratch_types=[
        pltpu.VMEM((NKC, CHUNK), jnp.int32),
        pltpu.VMEM((NKC, CHUNK), jnp.int32),
    ] + [pltpu.VMEM((CHUNK, D_H1), jnp.float32) for _ in range(NBUF)] + [
        pltpu.VMEM_SHARED((NPAD, D_H1), jnp.float32),
    ] + [pltpu.SemaphoreType.DMA for _ in range(2 * NBUF)],
    compiler_params=pltpu.CompilerParams(use_tc_tiling_on_sc=False),
)
def _sc_edge_agg(g_hbm, src_hbm, dst_hbm, zero_hbm, out_hbm,
                 sidx, didx, *rest):
    cid = lax.axis_index("c")
    sid = lax.axis_index("s")
    wid = sid * NC + cid
    rows = rest[:NBUF]
    acc = rest[NBUF]
    gsem = rest[NBUF + 1:2 * NBUF + 1]
    ssem = rest[2 * NBUF + 1:]

    @pl.when(sid == 0)
    def _():
        pltpu.sync_copy(zero_hbm, acc)

    base = wid * NKC
    pltpu.sync_copy(src_hbm.at[pl.ds(base, NKC)], sidx)
    pltpu.sync_copy(dst_hbm.at[pl.ds(base, NKC)], didx)
    plsc.subcore_barrier()

    for k in range(AHEAD):
        pltpu.async_copy(g_hbm.at[sidx.at[k]], rows[k % NBUF], gsem[k % NBUF])

    def body(i, carry):
        for j in range(NBUF):
            k = i * NBUF + j
            pltpu.make_async_copy(g_hbm.at[sidx.at[k]], rows[j],
                                  gsem[j]).wait()
            pltpu.async_copy(rows[j], acc.at[didx.at[k]], ssem[j], add=True)
            m = k + AHEAD
            jj = (j + AHEAD) % NBUF

            @pl.when((m >= NBUF) & (m < NKC))
            def _():
                pltpu.make_async_copy(rows[jj], acc.at[didx.at[0]],
                                      ssem[jj]).wait()

            @pl.when(m < NKC)
            def _():
                pltpu.async_copy(g_hbm.at[sidx.at[m]], rows[jj], gsem[jj])

        return carry

    lax.fori_loop(0, NKC // NBUF, body, 0)

    for j in range(NBUF):
        pltpu.make_async_copy(rows[j], acc.at[didx.at[0]], ssem[j]).wait()

    plsc.subcore_barrier()

    @pl.when(sid == 0)
    def _():
        pltpu.sync_copy(acc.at[pl.ds(0, N)], out_hbm.at[cid])


@functools.partial(
    pl.kernel,
    out_type=jax.ShapeDtypeStruct((NC, N, D_H1), jnp.float32),
    mesh=_sc_mesh,
    scratch_types=[
        pltpu.VMEM((NKC, CHUNK), jnp.int32),
        pltpu.VMEM((CHUNK, D_H1), jnp.float32),
        pltpu.VMEM_SHARED((NPAD, D_H1), jnp.float32),
        pltpu.SemaphoreType.DMA,
        pltpu.SemaphoreType.DMA,
        pltpu.SemaphoreType.DMA,
        pltpu.SemaphoreType.DMA,
    ],
    compiler_params=pltpu.CompilerParams(use_tc_tiling_on_sc=False),
)
def _sc_degree(dst_hbm, ones_hbm, zero_hbm, out_hbm, didx, ones_v, acc,
               ss0, ss1, ss2, ss3):
    cid = lax.axis_index("c")
    sid = lax.axis_index("s")
    wid = sid * NC + cid
    ssem = (ss0, ss1, ss2, ss3)
    DNB = 4

    @pl.when(sid == 0)
    def _():
        pltpu.sync_copy(zero_hbm, acc)

    pltpu.sync_copy(ones_hbm, ones_v)
    pltpu.sync_copy(dst_hbm.at[pl.ds(wid * NKC, NKC)], didx)
    plsc.subcore_barrier()

    def body(i, carry):
        for j in range(DNB):
            k = i * DNB + j

            @pl.when(k >= DNB)
            def _():
                pltpu.make_async_copy(ones_v, acc.at[didx.at[0]],
                                      ssem[j]).wait()

            pltpu.async_copy(ones_v, acc.at[didx.at[k]], ssem[j], add=True)
        return carry

    lax.fori_loop(0, NKC // DNB, body, 0)

    for j in range(DNB):
        pltpu.make_async_copy(ones_v, acc.at[didx.at[0]], ssem[j]).wait()

    plsc.subcore_barrier()

    @pl.when(sid == 0)
    def _():
        pltpu.sync_copy(acc.at[pl.ds(0, N)], out_hbm.at[cid])


def _tc_project1(x_ref, w1_ref, p1_ref):
    p1_ref[...] = jnp.dot(x_ref[...], w1_ref[...],
                          preferred_element_type=jnp.float32)


def _tc_scale1(degp_ref, p1_ref, dinv_ref, g1_ref):
    deg = degp_ref[0, :, :1] + degp_ref[1, :, :1] + 1.0
    dinv = lax.rsqrt(deg)
    dinv_ref[...] = dinv
    g1_ref[...] = dinv * p1_ref[...]


def _tc_mid(aggp_ref, p1_ref, dinv_ref, b1_ref, w2_ref, g2_ref, p2_ref):
    dinv = dinv_ref[...]
    agg = aggp_ref[0] + aggp_ref[1]
    h1 = jnp.maximum(dinv * agg + (dinv * dinv) * p1_ref[...] + b1_ref[...],
                     0.0)
    p2 = jnp.dot(h1, w2_ref[...], preferred_element_type=jnp.float32)
    p2_ref[...] = p2
    g2_ref[...] = jnp.concatenate(
        [dinv * p2, jnp.zeros((N, D_H1 - D_H2), jnp.float32)], axis=1)


def _tc_tail(aggp_ref, p2_ref, dinv_ref, b2_ref, batch_ref, ones_ref,
             w3_ref, b3_ref, out_ref):
    dinv = dinv_ref[...]
    agg = (aggp_ref[0] + aggp_ref[1])[:, :D_H2]
    h2 = jnp.maximum(dinv * agg + (dinv * dinv) * p2_ref[...] + b2_ref[...],
                     0.0)
    onehot_t = (lax.broadcasted_iota(jnp.int32, (G, N), 0)
                == batch_ref[...]).astype(jnp.float32)
    sums = jnp.dot(onehot_t, h2, preferred_element_type=jnp.float32)
    counts = jnp.dot(onehot_t, ones_ref[...],
                     preferred_element_type=jnp.float32)
    pooled = sums / jnp.maximum(counts, 1.0)
    out_ref[...] = jnp.dot(pooled, w3_ref[...],
                           preferred_element_type=jnp.float32) + b3_ref[...]


def kernel(x, edge_index, batch, W1, b1, W2, b2, W3, b3):
    npad_e = E_PAD - E
    src2 = jnp.concatenate(
        [edge_index[0], jnp.zeros((npad_e,), jnp.int32)]).reshape(-1, CHUNK)
    dst2 = jnp.concatenate(
        [edge_index[1], jnp.full((npad_e,), N, jnp.int32)]).reshape(-1, CHUNK)
    zero16 = jnp.zeros((NPAD, D_H1), jnp.float32)
    ones_chunk = jnp.ones((CHUNK, D_H1), jnp.float32)
    ones_col = jnp.ones((N, 1), jnp.float32)
    batch_row = batch.reshape(1, N)
    b1_2d = b1.reshape(1, D_H1)
    b2_2d = b2.reshape(1, D_H2)
    b3_2d = b3.reshape(1, 1)

    degp = _sc_degree(dst2, ones_chunk, zero16)

    p1 = pl.pallas_call(
        _tc_project1,
        out_shape=jax.ShapeDtypeStruct((N, D_H1), jnp.float32),
    )(x, W1)

    dinv, g1 = pl.pallas_call(
        _tc_scale1,
        out_shape=(jax.ShapeDtypeStruct((N, 1), jnp.float32),
                   jax.ShapeDtypeStruct((N, D_H1), jnp.float32)),
    )(degp, p1)

    agg1p = _sc_edge_agg(g1, src2, dst2, zero16)

    g2, p2 = pl.pallas_call(
        _tc_mid,
        out_shape=(jax.ShapeDtypeStruct((N, D_H1), jnp.float32),
                   jax.ShapeDtypeStruct((N, D_H2), jnp.float32)),
    )(agg1p, p1, dinv, b1_2d, W2)

    agg2p = _sc_edge_agg(g2, src2, dst2, zero16)

    out = pl.pallas_call(
        _tc_tail,
        out_shape=jax.ShapeDtypeStruct((G, 1), jnp.float32),
    )(agg2p, p2, dinv, b2_2d, batch_row, ones_col, W3, b3_2d)

    return out


# speedup vs baseline: 84.8657x; 1.8272x over previous
import functools

import jax
import jax.numpy as jnp
from jax import lax
from jax.experimental import pallas as pl
from jax.experimental.pallas import tpu as pltpu
from jax.experimental.pallas import tpu_sc as plsc

N = 10000
E = 640000
G = 64
D_IN = 121
D_H1 = 16
D_H2 = 8

NC = 2
NS = 16
NW = NC * NS
CHUNK = 128
NKC = 160
E_PAD = NW * NKC * CHUNK
NPAD = N + 8
NBUF = 8
AHEAD = 6

_sc_mesh = plsc.VectorSubcoreMesh(core_axis_name="c", subcore_axis_name="s")


@functools.partial(
    pl.kernel,
    out_type=jax.ShapeDtypeStruct((NC, N, D_H1), jnp.float32),
    mesh=_sc_mesh,
    scratch_types=[
        pltpu.VMEM((NKC, CHUNK), jnp.int32),
        pltpu.VMEM((NKC, CHUNK), jnp.int32),
    ] + [pltpu.VMEM((CHUNK, D_H1), jnp.float32) for _ in range(NBUF)] + [
        pltpu.VMEM_SHARED((NPAD, D_H1), jnp.float32),
        pltpu.VMEM_SHARED((NPAD, D_H1), jnp.float32),
    ] + [pltpu.SemaphoreType.DMA for _ in range(2 * NBUF)],
    compiler_params=pltpu.CompilerParams(use_tc_tiling_on_sc=False),
)
def _sc_edge_agg(g_hbm, src_hbm, dst_hbm, zero_hbm, out_hbm,
                 sidx, didx, *rest):
    cid = lax.axis_index("c")
    sid = lax.axis_index("s")
    wid = sid * NC + cid
    rows = rest[:NBUF]
    acc = rest[NBUF]
    gsp = rest[NBUF + 1]
    gsem = rest[NBUF + 2:NBUF + 2 + NBUF]
    ssem = rest[NBUF + 2 + NBUF:]

    @pl.when(sid == 0)
    def _():
        pltpu.sync_copy(zero_hbm, acc)

    @pl.when(sid == 1)
    def _():
        pltpu.sync_copy(g_hbm.at[pl.ds(0, N // 2)], gsp.at[pl.ds(0, N // 2)])

    @pl.when(sid == 2)
    def _():
        pltpu.sync_copy(g_hbm.at[pl.ds(N // 2, N // 2)],
                        gsp.at[pl.ds(N // 2, N // 2)])

    base = wid * NKC
    pltpu.sync_copy(src_hbm.at[pl.ds(base, NKC)], sidx)
    pltpu.sync_copy(dst_hbm.at[pl.ds(base, NKC)], didx)
    plsc.subcore_barrier()

    for k in range(AHEAD):
        pltpu.async_copy(gsp.at[sidx.at[k]], rows[k % NBUF], gsem[k % NBUF])

    def body(i, carry):
        for j in range(NBUF):
            k = i * NBUF + j
            pltpu.make_async_copy(gsp.at[sidx.at[k]], rows[j],
                                  gsem[j]).wait()
            pltpu.async_copy(rows[j], acc.at[didx.at[k]], ssem[j], add=True)
            m = k + AHEAD
            jj = (j + AHEAD) % NBUF

            @pl.when((m >= NBUF) & (m < NKC))
            def _():
                pltpu.make_async_copy(rows[jj], acc.at[didx.at[0]],
                                      ssem[jj]).wait()

            @pl.when(m < NKC)
            def _():
                pltpu.async_copy(gsp.at[sidx.at[m]], rows[jj], gsem[jj])

        return carry

    lax.fori_loop(0, NKC // NBUF, body, 0)

    for j in range(NBUF):
        pltpu.make_async_copy(rows[j], acc.at[didx.at[0]], ssem[j]).wait()

    plsc.subcore_barrier()

    @pl.when(sid == 0)
    def _():
        pltpu.sync_copy(acc.at[pl.ds(0, N)], out_hbm.at[cid])


@functools.partial(
    pl.kernel,
    out_type=jax.ShapeDtypeStruct((NC, N, D_H1), jnp.float32),
    mesh=_sc_mesh,
    scratch_types=[
        pltpu.VMEM((NKC, CHUNK), jnp.int32),
        pltpu.VMEM((CHUNK, D_H1), jnp.float32),
        pltpu.VMEM_SHARED((NPAD, D_H1), jnp.float32),
        pltpu.SemaphoreType.DMA,
        pltpu.SemaphoreType.DMA,
        pltpu.SemaphoreType.DMA,
        pltpu.SemaphoreType.DMA,
    ],
    compiler_params=pltpu.CompilerParams(use_tc_tiling_on_sc=False),
)
def _sc_degree(dst_hbm, ones_hbm, zero_hbm, out_hbm, didx, ones_v, acc,
               ss0, ss1, ss2, ss3):
    cid = lax.axis_index("c")
    sid = lax.axis_index("s")
    wid = sid * NC + cid
    ssem = (ss0, ss1, ss2, ss3)
    DNB = 4

    @pl.when(sid == 0)
    def _():
        pltpu.sync_copy(zero_hbm, acc)

    pltpu.sync_copy(ones_hbm, ones_v)
    pltpu.sync_copy(dst_hbm.at[pl.ds(wid * NKC, NKC)], didx)
    plsc.subcore_barrier()

    def body(i, carry):
        for j in range(DNB):
            k = i * DNB + j

            @pl.when(k >= DNB)
            def _():
                pltpu.make_async_copy(ones_v, acc.at[didx.at[0]],
                                      ssem[j]).wait()

            pltpu.async_copy(ones_v, acc.at[didx.at[k]], ssem[j], add=True)
        return carry

    lax.fori_loop(0, NKC // DNB, body, 0)

    for j in range(DNB):
        pltpu.make_async_copy(ones_v, acc.at[didx.at[0]], ssem[j]).wait()

    plsc.subcore_barrier()

    @pl.when(sid == 0)
    def _():
        pltpu.sync_copy(acc.at[pl.ds(0, N)], out_hbm.at[cid])


def _tc_project1(x_ref, w1_ref, p1_ref):
    p1_ref[...] = jnp.dot(x_ref[...], w1_ref[...],
                          preferred_element_type=jnp.float32)


def _tc_scale1(degp_ref, p1_ref, dinv_ref, g1_ref):
    deg = degp_ref[0, :, :1] + degp_ref[1, :, :1] + 1.0
    dinv = lax.rsqrt(deg)
    dinv_ref[...] = dinv
    g1_ref[...] = dinv * p1_ref[...]


def _tc_mid(aggp_ref, p1_ref, dinv_ref, b1_ref, w2_ref, g2_ref, p2_ref):
    dinv = dinv_ref[...]
    agg = aggp_ref[0] + aggp_ref[1]
    h1 = jnp.maximum(dinv * agg + (dinv * dinv) * p1_ref[...] + b1_ref[...],
                     0.0)
    p2 = jnp.dot(h1, w2_ref[...], preferred_element_type=jnp.float32)
    p2_ref[...] = p2
    g2_ref[...] = jnp.concatenate(
        [dinv * p2, jnp.zeros((N, D_H1 - D_H2), jnp.float32)], axis=1)


def _tc_tail(aggp_ref, p2_ref, dinv_ref, b2_ref, batch_ref, ones_ref,
             w3_ref, b3_ref, out_ref):
    dinv = dinv_ref[...]
    agg = (aggp_ref[0] + aggp_ref[1])[:, :D_H2]
    h2 = jnp.maximum(dinv * agg + (dinv * dinv) * p2_ref[...] + b2_ref[...],
                     0.0)
    onehot_t = (lax.broadcasted_iota(jnp.int32, (G, N), 0)
                == batch_ref[...]).astype(jnp.float32)
    sums = jnp.dot(onehot_t, h2, preferred_element_type=jnp.float32)
    counts = jnp.dot(onehot_t, ones_ref[...],
                     preferred_element_type=jnp.float32)
    pooled = sums / jnp.maximum(counts, 1.0)
    out_ref[...] = jnp.dot(pooled, w3_ref[...],
                           preferred_element_type=jnp.float32) + b3_ref[...]


def kernel(x, edge_index, batch, W1, b1, W2, b2, W3, b3):
    npad_e = E_PAD - E
    src2 = jnp.concatenate(
        [edge_index[0], jnp.zeros((npad_e,), jnp.int32)]).reshape(-1, CHUNK)
    dst2 = jnp.concatenate(
        [edge_index[1], jnp.full((npad_e,), N, jnp.int32)]).reshape(-1, CHUNK)
    zero16 = jnp.zeros((NPAD, D_H1), jnp.float32)
    ones_chunk = jnp.ones((CHUNK, D_H1), jnp.float32)
    ones_col = jnp.ones((N, 1), jnp.float32)
    batch_row = batch.reshape(1, N)
    b1_2d = b1.reshape(1, D_H1)
    b2_2d = b2.reshape(1, D_H2)
    b3_2d = b3.reshape(1, 1)

    degp = _sc_degree(dst2, ones_chunk, zero16)

    p1 = pl.pallas_call(
        _tc_project1,
        out_shape=jax.ShapeDtypeStruct((N, D_H1), jnp.float32),
    )(x, W1)

    dinv, g1 = pl.pallas_call(
        _tc_scale1,
        out_shape=(jax.ShapeDtypeStruct((N, 1), jnp.float32),
                   jax.ShapeDtypeStruct((N, D_H1), jnp.float32)),
    )(degp, p1)

    agg1p = _sc_edge_agg(g1, src2, dst2, zero16)

    g2, p2 = pl.pallas_call(
        _tc_mid,
        out_shape=(jax.ShapeDtypeStruct((N, D_H1), jnp.float32),
                   jax.ShapeDtypeStruct((N, D_H2), jnp.float32)),
    )(agg1p, p1, dinv, b1_2d, W2)

    agg2p = _sc_edge_agg(g2, src2, dst2, zero16)

    out = pl.pallas_call(
        _tc_tail,
        out_shape=jax.ShapeDtypeStruct((G, 1), jnp.float32),
    )(agg2p, p2, dinv, b2_2d, batch_row, ones_col, W3, b3_2d)

    return out


# speedup vs baseline: 101.3782x; 1.1946x over previous
import functools

import jax
import jax.numpy as jnp
from jax import lax
from jax.experimental import pallas as pl
from jax.experimental.pallas import tpu as pltpu
from jax.experimental.pallas import tpu_sc as plsc

N = 10000
E = 640000
G = 64
D_IN = 121
D_H1 = 16
D_H2 = 8

NC = 2
NS = 16
NW = NC * NS
CHUNK = 128
NCHUNK = E // CHUNK
NKC = NCHUNK // NW
NEXTRA = NCHUNK - NW * NKC
NPAD = N + 8
NBUF = 6
AHEAD = 4

_sc_mesh = plsc.VectorSubcoreMesh(core_axis_name="c", subcore_axis_name="s")


@functools.partial(
    pl.kernel,
    out_type=jax.ShapeDtypeStruct((NC, N, D_H1), jnp.float32),
    mesh=_sc_mesh,
    scratch_types=[
        pltpu.VMEM((NKC + 1, CHUNK), jnp.int32),
        pltpu.VMEM((NKC + 1, CHUNK), jnp.int32),
    ] + [pltpu.VMEM((CHUNK, D_H1), jnp.float32) for _ in range(NBUF)] + [
        pltpu.VMEM_SHARED((NPAD, D_H1), jnp.float32),
        pltpu.VMEM_SHARED((NPAD, D_H1), jnp.float32),
    ] + [pltpu.SemaphoreType.DMA for _ in range(2 * NBUF)],
    compiler_params=pltpu.CompilerParams(use_tc_tiling_on_sc=False),
)
def _sc_edge_agg(g_hbm, edges_hbm, zero_hbm, out_hbm,
                 sidx, didx, *rest):
    cid = lax.axis_index("c")
    sid = lax.axis_index("s")
    wid = sid * NC + cid
    rows = rest[:NBUF]
    acc = rest[NBUF]
    gsp = rest[NBUF + 1]
    gsem = rest[NBUF + 2:NBUF + 2 + NBUF]
    ssem = rest[NBUF + 2 + NBUF:]

    @pl.when(sid == 0)
    def _():
        pltpu.sync_copy(zero_hbm, acc)

    @pl.when(sid == 1)
    def _():
        pltpu.sync_copy(g_hbm.at[pl.ds(0, N // 2)], gsp.at[pl.ds(0, N // 2)])

    @pl.when(sid == 2)
    def _():
        pltpu.sync_copy(g_hbm.at[pl.ds(N // 2, N // 2)],
                        gsp.at[pl.ds(N // 2, N // 2)])

    base = wid * NKC
    pltpu.sync_copy(edges_hbm.at[0, pl.ds(base, NKC)],
                    sidx.at[pl.ds(0, NKC)])
    pltpu.sync_copy(edges_hbm.at[1, pl.ds(base, NKC)],
                    didx.at[pl.ds(0, NKC)])

    @pl.when(wid < NEXTRA)
    def _():
        xb = NW * NKC + wid
        pltpu.sync_copy(edges_hbm.at[0, pl.ds(xb, 1)],
                        sidx.at[pl.ds(NKC, 1)])
        pltpu.sync_copy(edges_hbm.at[1, pl.ds(xb, 1)],
                        didx.at[pl.ds(NKC, 1)])

    plsc.subcore_barrier()

    for k in range(AHEAD):
        pltpu.async_copy(gsp.at[sidx.at[k]], rows[k % NBUF], gsem[k % NBUF])

    def body(i, carry):
        for j in range(NBUF):
            k = i * NBUF + j
            pltpu.make_async_copy(gsp.at[sidx.at[k]], rows[j],
                                  gsem[j]).wait()
            pltpu.async_copy(rows[j], acc.at[didx.at[k]], ssem[j], add=True)
            m = k + AHEAD
            jj = (j + AHEAD) % NBUF

            @pl.when((m >= NBUF) & (m < NKC))
            def _():
                pltpu.make_async_copy(rows[jj], acc.at[didx.at[0]],
                                      ssem[jj]).wait()

            @pl.when(m < NKC)
            def _():
                pltpu.async_copy(gsp.at[sidx.at[m]], rows[jj], gsem[jj])

        return carry

    lax.fori_loop(0, NKC // NBUF, body, 0)

    for j in range(NBUF):
        pltpu.make_async_copy(rows[j], acc.at[didx.at[0]], ssem[j]).wait()

    @pl.when(wid < NEXTRA)
    def _():
        pltpu.async_copy(gsp.at[sidx.at[NKC]], rows[0], gsem[0])
        pltpu.make_async_copy(gsp.at[sidx.at[NKC]], rows[0], gsem[0]).wait()
        pltpu.async_copy(rows[0], acc.at[didx.at[NKC]], ssem[0], add=True)
        pltpu.make_async_copy(rows[0], acc.at[didx.at[NKC]], ssem[0]).wait()

    plsc.subcore_barrier()

    @pl.when(sid == 0)
    def _():
        pltpu.sync_copy(acc.at[pl.ds(0, N)], out_hbm.at[cid])


@functools.partial(
    pl.kernel,
    out_type=jax.ShapeDtypeStruct((NC, N, D_H1), jnp.float32),
    mesh=_sc_mesh,
    scratch_types=[
        pltpu.VMEM((NKC + 1, CHUNK), jnp.int32),
        pltpu.VMEM((CHUNK, D_H1), jnp.float32),
        pltpu.VMEM_SHARED((NPAD, D_H1), jnp.float32),
        pltpu.SemaphoreType.DMA,
        pltpu.SemaphoreType.DMA,
        pltpu.SemaphoreType.DMA,
        pltpu.SemaphoreType.DMA,
    ],
    compiler_params=pltpu.CompilerParams(use_tc_tiling_on_sc=False),
)
def _sc_degree(edges_hbm, ones_hbm, zero_hbm, out_hbm, didx, ones_v, acc,
               ss0, ss1, ss2, ss3):
    cid = lax.axis_index("c")
    sid = lax.axis_index("s")
    wid = sid * NC + cid
    ssem = (ss0, ss1, ss2, ss3)
    DNB = 4

    @pl.when(sid == 0)
    def _():
        pltpu.sync_copy(zero_hbm, acc)

    pltpu.sync_copy(ones_hbm, ones_v)
    pltpu.sync_copy(edges_hbm.at[1, pl.ds(wid * NKC, NKC)],
                    didx.at[pl.ds(0, NKC)])

    @pl.when(wid < NEXTRA)
    def _():
        pltpu.sync_copy(edges_hbm.at[1, pl.ds(NW * NKC + wid, 1)],
                        didx.at[pl.ds(NKC, 1)])

    plsc.subcore_barrier()

    def body(i, carry):
        for j in range(DNB):
            k = i * DNB + j

            @pl.when(k >= DNB)
            def _():
                pltpu.make_async_copy(ones_v, acc.at[didx.at[0]],
                                      ssem[j]).wait()

            pltpu.async_copy(ones_v, acc.at[didx.at[k]], ssem[j], add=True)
        return carry

    lax.fori_loop(0, NKC // DNB, body, 0)

    for j in range(DNB):
        pltpu.make_async_copy(ones_v, acc.at[didx.at[0]], ssem[j]).wait()

    @pl.when(wid < NEXTRA)
    def _():
        pltpu.async_copy(ones_v, acc.at[didx.at[NKC]], ssem[0], add=True)
        pltpu.make_async_copy(ones_v, acc.at[didx.at[NKC]], ssem[0]).wait()

    plsc.subcore_barrier()

    @pl.when(sid == 0)
    def _():
        pltpu.sync_copy(acc.at[pl.ds(0, N)], out_hbm.at[cid])


def _tc_project1(x_ref, w1_ref, p1_ref):
    p1_ref[...] = jnp.dot(x_ref[...], w1_ref[...],
                          preferred_element_type=jnp.float32)


def _tc_scale1(degp_ref, p1_ref, dinv_ref, g1_ref):
    deg = degp_ref[0, :, :1] + degp_ref[1, :, :1] + 1.0
    dinv = lax.rsqrt(deg)
    dinv_ref[...] = dinv
    g1_ref[...] = dinv * p1_ref[...]


def _tc_mid(aggp_ref, p1_ref, dinv_ref, b1_ref, w2_ref, g2_ref, p2_ref):
    dinv = dinv_ref[...]
    agg = aggp_ref[0] + aggp_ref[1]
    h1 = jnp.maximum(dinv * agg + (dinv * dinv) * p1_ref[...] + b1_ref[...],
                     0.0)
    p2 = jnp.dot(h1, w2_ref[...], preferred_element_type=jnp.float32)
    p2_ref[...] = p2
    g2_ref[...] = jnp.concatenate(
        [dinv * p2, jnp.zeros((N, D_H1 - D_H2), jnp.float32)], axis=1)


def _tc_tail(aggp_ref, p2_ref, dinv_ref, b2_ref, batch_ref, ones_ref,
             w3_ref, b3_ref, out_ref):
    dinv = dinv_ref[...]
    agg = (aggp_ref[0] + aggp_ref[1])[:, :D_H2]
    h2 = jnp.maximum(dinv * agg + (dinv * dinv) * p2_ref[...] + b2_ref[...],
                     0.0)
    onehot_t = (lax.broadcasted_iota(jnp.int32, (G, N), 0)
                == batch_ref[...]).astype(jnp.float32)
    sums = jnp.dot(onehot_t, h2, preferred_element_type=jnp.float32)
    counts = jnp.dot(onehot_t, ones_ref[...],
                     preferred_element_type=jnp.float32)
    pooled = sums / jnp.maximum(counts, 1.0)
    out_ref[...] = jnp.dot(pooled, w3_ref[...],
                           preferred_element_type=jnp.float32) + b3_ref[...]


def kernel(x, edge_index, batch, W1, b1, W2, b2, W3, b3):
    edges = edge_index.reshape(2, NCHUNK, CHUNK)
    zero16 = jnp.zeros((NPAD, D_H1), jnp.float32)
    ones_chunk = jnp.ones((CHUNK, D_H1), jnp.float32)
    ones_col = jnp.ones((N, 1), jnp.float32)
    batch_row = batch.reshape(1, N)
    b1_2d = b1.reshape(1, D_H1)
    b2_2d = b2.reshape(1, D_H2)
    b3_2d = b3.reshape(1, 1)

    degp = _sc_degree(edges, ones_chunk, zero16)

    p1 = pl.pallas_call(
        _tc_project1,
        out_shape=jax.ShapeDtypeStruct((N, D_H1), jnp.float32),
    )(x, W1)

    dinv, g1 = pl.pallas_call(
        _tc_scale1,
        out_shape=(jax.ShapeDtypeStruct((N, 1), jnp.float32),
                   jax.ShapeDtypeStruct((N, D_H1), jnp.float32)),
    )(degp, p1)

    agg1p = _sc_edge_agg(g1, edges, zero16)

    g2, p2 = pl.pallas_call(
        _tc_mid,
        out_shape=(jax.ShapeDtypeStruct((N, D_H1), jnp.float32),
                   jax.ShapeDtypeStruct((N, D_H2), jnp.float32)),
    )(agg1p, p1, dinv, b1_2d, W2)

    agg2p = _sc_edge_agg(g2, edges, zero16)

    out = pl.pallas_call(
        _tc_tail,
        out_shape=jax.ShapeDtypeStruct((G, 1), jnp.float32),
    )(agg2p, p2, dinv, b2_2d, batch_row, ones_col, W3, b3_2d)

    return out
